# baseline (speedup 1.0000x reference)
import functools

import jax
import jax.numpy as jnp
from jax import lax
from jax.experimental import pallas as pl
from jax.experimental.pallas import tpu as pltpu
from jax.experimental.pallas import tpu_sc as plsc

_B = 4


def _scs_gather_bcast(idx, table):
    n_sel = idx.shape[0]
    n_pool, length, dim = table.shape
    info = plsc.get_sparse_core_info()
    nc = info.num_cores
    per_c = n_sel // nc

    mesh = plsc.ScalarSubcoreMesh(axis_name="c", num_cores=nc)

    @functools.partial(
        pl.kernel,
        mesh=mesh,
        out_type=jax.ShapeDtypeStruct((_B, n_sel * length, 1, dim), jnp.float32),
        scratch_types=[
            pltpu.SMEM((n_sel,), jnp.int32),
            pltpu.SemaphoreType.DMA,
        ],
    )
    def body(idx_hbm, table_hbm, out_hbm, idx_s, sem):
        cid = lax.axis_index("c")
        base = cid * per_c
        pltpu.sync_copy(idx_hbm, idx_s)

        def issue(i, carry):
            sel = base + i
            v = idx_s[sel]
            for b in range(_B):
                pltpu.async_copy(
                    table_hbm.at[v],
                    out_hbm.at[b, pl.ds(sel * length, length), 0],
                    sem,
                ).wait()
            return carry

        lax.fori_loop(0, per_c, issue, 0)

    return body(idx, table)


def kernel(indices, batch_size, prompts):
    del batch_size
    return _scs_gather_bcast(indices.astype(jnp.int32), prompts)
